# baseline (device time: 8252 ns/iter reference)
import jax
import jax.numpy as jnp
from jax import lax
from jax.experimental import pallas as pl
from jax.experimental.pallas import tpu as pltpu

N_DEV = 4


def _cmp_stage(v, rows, j, take_max):
    m = v.shape[0]
    down = pltpu.roll(v, m - j, 0)
    up = pltpu.roll(v, j, 0)
    w = jnp.where((rows & j) == 0, down, up)
    return jnp.where(take_max, jnp.maximum(v, w), jnp.minimum(v, w))


def kernel(x):
    m_per, n = x.shape
    m_total = N_DEV * m_per
    log_per = m_per.bit_length() - 1
    log_tot = m_total.bit_length() - 1
    assert (1 << log_per) == m_per and (1 << log_tot) == m_total

    def body(x_ref, out_ref, gbuf, sorted_ref, send_sems, recv_sems):
        my = lax.axis_index("i")
        my_odd = (my % 2) == 1

        barrier_sem = pltpu.get_barrier_semaphore()
        for d in range(1, N_DEV):
            pl.semaphore_signal(
                barrier_sem,
                inc=1,
                device_id=((my + d) % N_DEV,),
                device_id_type=pl.DeviceIdType.MESH,
            )

        v = x_ref[...].astype(jnp.bfloat16)
        rows_l = lax.broadcasted_iota(jnp.int32, (m_per, n), 0)
        for s in range(log_per):
            kk = 1 << (s + 1)
            j = kk >> 1
            while j >= 1:
                tm = ((rows_l & j) != 0) != ((rows_l & kk) != 0)
                if kk == m_per:
                    tm = tm != my_odd
                v = _cmp_stage(v, rows_l, j, tm)
                j >>= 1
        gbuf[my] = v

        pl.semaphore_wait(barrier_sem, N_DEV - 1)

        rdmas = []
        for d in range(1, N_DEV):
            rdma = pltpu.make_async_remote_copy(
                src_ref=gbuf.at[my],
                dst_ref=gbuf.at[my],
                send_sem=send_sems.at[d - 1],
                recv_sem=recv_sems.at[d - 1],
                device_id=((my + d) % N_DEV,),
                device_id_type=pl.DeviceIdType.MESH,
            )
            rdma.start()
            rdmas.append(rdma)
        for rdma in rdmas:
            rdma.wait()

        v = jnp.concatenate([gbuf[d] for d in range(N_DEV)], axis=0)
        rows = lax.broadcasted_iota(jnp.int32, (m_total, n), 0)
        for s in range(log_per, log_tot):
            kk = 1 << (s + 1)
            j = kk >> 1
            while j >= 1:
                tm = ((rows & j) != 0) != ((rows & kk) != 0)
                v = _cmp_stage(v, rows, j, tm)
                j >>= 1

        sorted_ref[...] = v.astype(jnp.float32)
        out_ref[...] = sorted_ref[pl.ds(my * m_per, m_per), :]

    return pl.pallas_call(
        body,
        out_shape=jax.ShapeDtypeStruct((m_per, n), jnp.float32),
        in_specs=[pl.BlockSpec(memory_space=pltpu.VMEM)],
        out_specs=pl.BlockSpec(memory_space=pltpu.VMEM),
        scratch_shapes=[
            pltpu.VMEM((N_DEV, m_per, n), jnp.bfloat16),
            pltpu.VMEM((m_total, n), jnp.float32),
            pltpu.SemaphoreType.DMA((N_DEV - 1,)),
            pltpu.SemaphoreType.DMA((N_DEV - 1,)),
        ],
        compiler_params=pltpu.CompilerParams(collective_id=0),
    )(x)


# device time: 7768 ns/iter; 1.0623x vs baseline; 1.0623x over previous
import jax
import jax.numpy as jnp
from jax import lax
from jax.experimental import pallas as pl
from jax.experimental.pallas import tpu as pltpu

N_DEV = 4


def _cmp_stage(v, rows, j, take_max):
    m = v.shape[0]
    down = pltpu.roll(v, m - j, 0)
    up = pltpu.roll(v, j, 0)
    w = jnp.where((rows & j) == 0, down, up)
    return jnp.where(take_max, jnp.maximum(v, w), jnp.minimum(v, w))


def kernel(x):
    m_per, n = x.shape
    m_total = N_DEV * m_per
    m_pair = 2 * m_per
    log_per = m_per.bit_length() - 1
    assert (1 << log_per) == m_per

    def body(xt_hbm, out_ref, gbuf, xv, local_sems, send_sems, recv_sems):
        my = lax.axis_index("i")
        my_odd = (my % 2) == 1
        in_pair_b = my >= 2

        barrier_sem = pltpu.get_barrier_semaphore()
        for d in range(1, N_DEV):
            pl.semaphore_signal(
                barrier_sem,
                inc=1,
                device_id=((my + d) % N_DEV,),
                device_id_type=pl.DeviceIdType.MESH,
            )

        load = pltpu.make_async_copy(xt_hbm, xv, local_sems.at[0])
        load.start()
        load.wait()

        v = jnp.transpose(xv[...]).astype(jnp.bfloat16)
        rows_l = lax.broadcasted_iota(jnp.int32, (m_per, n), 0)
        for s in range(log_per):
            kk = 1 << (s + 1)
            j = kk >> 1
            while j >= 1:
                tm = ((rows_l & j) != 0) != ((rows_l & kk) != 0)
                if kk == m_per:
                    tm = tm != my_odd
                v = _cmp_stage(v, rows_l, j, tm)
                j >>= 1
        my_rows = pl.ds(my * m_per, m_per)
        gbuf[my_rows, :] = v

        pl.semaphore_wait(barrier_sem, N_DEV - 1)

        targets = [my ^ 1, 3 - my, (my + 2) % N_DEV]
        rdmas = []
        for rel, tgt in enumerate(targets):
            rdma = pltpu.make_async_remote_copy(
                src_ref=gbuf.at[my_rows, :],
                dst_ref=gbuf.at[my_rows, :],
                send_sem=send_sems.at[rel],
                recv_sem=recv_sems.at[rel],
                device_id=(tgt,),
                device_id_type=pl.DeviceIdType.MESH,
            )
            rdma.start()
            rdmas.append(rdma)

        rows_p = lax.broadcasted_iota(jnp.int32, (m_pair, n), 0)

        def merge_pair(vp, desc):
            j = m_per
            while j >= 1:
                tm = ((rows_p & j) != 0) != desc
                vp = _cmp_stage(vp, rows_p, j, tm)
                j >>= 1
            return vp

        rdmas[0].wait_recv()
        partner = gbuf[pl.ds((my ^ 1) * m_per, m_per), :]
        vp = jnp.concatenate(
            [jnp.where(my_odd, partner, v), jnp.where(my_odd, v, partner)],
            axis=0,
        )
        vp = merge_pair(vp, in_pair_b)

        rdmas[1].wait_recv()
        rdmas[2].wait_recv()
        other_base = jnp.where(in_pair_b, 0, m_pair)
        other = gbuf[pl.ds(other_base, m_pair), :]
        other = merge_pair(other, jnp.logical_not(in_pair_b))

        half = jnp.where(
            in_pair_b, jnp.maximum(vp, other), jnp.minimum(vp, other)
        )
        blk = jnp.where(
            my_odd,
            jnp.maximum(half[:m_per], half[m_per:]),
            jnp.minimum(half[:m_per], half[m_per:]),
        )
        j = m_per >> 1
        while j >= 1:
            blk = _cmp_stage(blk, rows_l, j, (rows_l & j) != 0)
            j >>= 1

        out_ref[...] = jnp.transpose(blk.astype(jnp.float32))
        for rdma in rdmas:
            rdma.wait_send()

    out_t = pl.pallas_call(
        body,
        out_shape=jax.ShapeDtypeStruct((n, m_per), jnp.float32),
        in_specs=[pl.BlockSpec(memory_space=pltpu.MemorySpace.HBM)],
        out_specs=pl.BlockSpec(memory_space=pltpu.VMEM),
        scratch_shapes=[
            pltpu.VMEM((m_total, n), jnp.bfloat16),
            pltpu.VMEM((n, m_per), jnp.float32),
            pltpu.SemaphoreType.DMA((1,)),
            pltpu.SemaphoreType.DMA((N_DEV - 1,)),
            pltpu.SemaphoreType.DMA((N_DEV - 1,)),
        ],
        compiler_params=pltpu.CompilerParams(collective_id=0),
    )(pltpu.with_memory_space_constraint(x.T, pltpu.MemorySpace.HBM))
    return out_t.T
